# baseline (speedup 1.0000x reference)
import functools

import jax
import jax.numpy as jnp
from jax import lax
from jax.experimental import pallas as pl
from jax.experimental.pallas import tpu as pltpu
from jax.experimental.pallas import tpu_sc as plsc

_N = 128
_M = 100000
_T = 16384
_MT = 2048
_GRID = -(-_M // _MT)

_NW = 32
_BPW = _T // _NW
_NCH = _BPW // 128
_NV = _N // 16


def _tc_body(a_ref, at_ref, lse_ref, acc_ref):
    i = pl.program_id(0)
    x = a_ref[...]
    col = i * _MT + lax.broadcasted_iota(jnp.int32, x.shape, 1)
    e = jnp.where(col < _M, jnp.exp(x), 0.0)
    partial = jnp.sum(e, axis=1, keepdims=True)

    @pl.when(i == 0)
    def _init():
        acc_ref[...] = jnp.zeros_like(acc_ref)

    acc_ref[...] += partial
    at_ref[...] = x.T

    @pl.when(i == _GRID - 1)
    def _fin():
        lse_ref[...] = jnp.log(acc_ref[...])


def _tc_pass(a):
    return pl.pallas_call(
        _tc_body,
        grid=(_GRID,),
        in_specs=[pl.BlockSpec((_N, _MT), lambda i: (0, i))],
        out_specs=[
            pl.BlockSpec((_MT, _N), lambda i: (i, 0)),
            pl.BlockSpec((_N, 1), lambda i: (0, 0)),
        ],
        out_shape=[
            jax.ShapeDtypeStruct((_M, _N), jnp.float32),
            jax.ShapeDtypeStruct((_N, 1), jnp.float32),
        ],
        scratch_shapes=[pltpu.VMEM((_N, 1), jnp.float32)],
    )(a)


@functools.cache
def _make_sc_gather():
    mesh = plsc.VectorSubcoreMesh(core_axis_name="c", subcore_axis_name="s")
    return pl.kernel(
        _sc_gather_body,
        mesh=mesh,
        out_type=jax.ShapeDtypeStruct((_T, _N), jnp.float32),
        scratch_types=[
            pltpu.VMEM((_NCH, 128), jnp.int32),
            pltpu.VMEM((_BPW, _N), jnp.float32),
            pltpu.VMEM((_N,), jnp.float32),
            pltpu.SemaphoreType.DMA,
        ],
    )


def _sc_gather_body(at_hbm, obs_hbm, lse_hbm, out_hbm, idx_v, rows_v, lse_v, sem):
    wid = lax.axis_index("s") * 2 + lax.axis_index("c")
    base = wid * _BPW
    pltpu.sync_copy(obs_hbm.at[pl.ds(wid * _NCH, _NCH)], idx_v)
    pltpu.sync_copy(lse_hbm, lse_v)
    copies = [
        pltpu.async_copy(
            at_hbm.at[idx_v.at[j]], rows_v.at[pl.ds(j * 128, 128)], sem
        )
        for j in range(_NCH)
    ]
    for c in copies:
        c.wait()
    lvs = [lse_v[pl.ds(16 * j, 16)] for j in range(_NV)]

    def body(i, carry):
        for j in range(_NV):
            sl = pl.ds(16 * j, 16)
            rows_v[i, sl] = rows_v[i, sl] - lvs[j]
        return carry

    lax.fori_loop(0, _BPW, body, 0)
    pltpu.sync_copy(rows_v, out_hbm.at[pl.ds(base, _BPW)])


def kernel(obervation_raw, unnormalized_emission_matrix):
    obs2 = obervation_raw.astype(jnp.int32).reshape(_T // 128, 128)
    at, lse2 = _tc_pass(unnormalized_emission_matrix)
    return _make_sc_gather()(at, obs2, lse2.reshape(_N))
